# baseline (speedup 1.0000x reference)
import functools

import jax
import jax.numpy as jnp
from jax.experimental import pallas as pl
from jax.experimental.pallas import tpu as pltpu

N = 50000
E = 800000
B = 2048
G = 16

NBLK = 2000
EBLK = 3200
RBLK = 1000
EPS = 1e-12


def _leaky(x):
    return jnp.where(x >= 0, x, 0.01 * x)


def _sigmoid(x):
    return 1.0 / (1.0 + jnp.exp(-x))


def _elu(x):
    return jnp.where(x > 0, x, jnp.exp(x) - 1.0)


def _mm(a, b):
    return jax.lax.dot_general(a, b, (((1,), (0,)), ((), ())),
                               preferred_element_type=jnp.float32)


def _k1(nf_ref, Wpn_ref, bpn_ref, Wpa_ref, u_ref, o_hv, o_P, o_D):
    nf = nf_ref[...]
    hv = _leaky(_mm(nf, Wpn_ref[...]) + bpn_ref[...])
    o_hv[...] = hv
    o_P[...] = _mm(nf, Wpa_ref[...])
    o_D[...] = _mm(hv, u_ref[...])


def _k2(Ps_ref, Dd_ref, ef_ref, Wpb_ref, bpe1_ref, vv_ref, o_m, o_x):
    he1 = _leaky(Ps_ref[...] + _mm(ef_ref[...], Wpb_ref[...]) + bpe1_ref[...])
    lg = Dd_ref[...] + _mm(he1, vv_ref[...])
    ex = jnp.exp(_leaky(lg))
    o_m[...] = ex * he1
    o_x[...] = ex


def _k4(t_ref, s_ref, hv_ref, Wet_ref, bet_ref, Wg_ref, bg_ref, Wpr_ref,
        bpr_ref, c2_ref, o_hf, o_hv2, o_zn):
    s = s_ref[...]
    occ = s / (s + EPS)
    c = _mm(t_ref[...] / (s + EPS), Wet_ref[...]) + bet_ref[...] * occ
    xg = _elu(c)
    h = hv_ref[...]
    gi = _mm(xg, Wg_ref[:, 0:48])
    gh = _mm(h, Wg_ref[:, 48:96])
    bg = bg_ref[...]
    r = _sigmoid(gi[:, 0:16] + gh[:, 0:16] + bg[:, 0:16])
    z = _sigmoid(gi[:, 16:32] + gh[:, 16:32] + bg[:, 16:32])
    n = jnp.tanh(gi[:, 32:48] + bg[:, 32:48] + r * (gh[:, 32:48] + bg[:, 48:64]))
    hf = jnp.maximum((1.0 - z) * n + z * h, 0.0)
    o_hf[...] = hf
    o_hv2[...] = _mm(hf, Wpr_ref[...]) + bpr_ref[...]
    o_zn[...] = _mm(hf, c2_ref[...])


def _k5(hf_ref, gid_ref, c1_ref, bcl_ref, o_gf, o_dg):
    i = pl.program_id(0)
    gid = gid_ref[...]
    iota = jax.lax.broadcasted_iota(jnp.int32, (RBLK, B), 1)
    oh = jnp.where(gid == iota, 1.0, 0.0)
    contrib = jax.lax.dot_general(oh, hf_ref[...], (((0,), (0,)), ((), ())),
                                  preferred_element_type=jnp.float32)

    @pl.when(i == 0)
    def _():
        o_gf[...] = jnp.zeros_like(o_gf)
        o_dg[...] = jnp.zeros_like(o_dg)

    o_gf[...] += contrib

    @pl.when(i == pl.num_programs(0) - 1)
    def _():
        gf = o_gf[...]
        o_dg[...] = _mm(jnp.maximum(gf, 0.0), c1_ref[...]) + bcl_ref[0, 0]


def _k6(hv_ref, zn_ref, gid_ref, dg_ref, o_t2, o_s2):
    i = pl.program_id(0)
    gid = gid_ref[...]
    iota = jax.lax.broadcasted_iota(jnp.int32, (RBLK, B), 1)
    oh = jnp.where(gid == iota, 1.0, 0.0)
    dgn = _mm(oh, dg_ref[...])
    zl = _leaky(dgn + zn_ref[...])
    ex2 = jnp.exp(zl)
    ct = jax.lax.dot_general(oh, ex2 * hv_ref[...], (((0,), (0,)), ((), ())),
                             preferred_element_type=jnp.float32)
    cs = jax.lax.dot_general(oh, ex2, (((0,), (0,)), ((), ())),
                             preferred_element_type=jnp.float32)

    @pl.when(i == 0)
    def _():
        o_t2[...] = jnp.zeros_like(o_t2)
        o_s2[...] = jnp.zeros_like(o_s2)

    o_t2[...] += ct
    o_s2[...] += cs


def _k7(t21_ref, s21_ref, gf1_ref, t22_ref, s22_ref, gf2_ref, Wr_ref, br_ref,
        x_ref, W_ref, b_ref, emb_ref, o_ref):
    def graph_branch(t2, s2, gf):
        grepr = _elu(t2 / (s2 + EPS))
        gi = _mm(grepr, Wr_ref[:, 0:48])
        gh = _mm(gf, Wr_ref[:, 48:96])
        br = br_ref[...]
        r = _sigmoid(gi[:, 0:16] + gh[:, 0:16] + br[:, 0:16])
        z = _sigmoid(gi[:, 16:32] + gh[:, 16:32] + br[:, 16:32])
        n = jnp.tanh(gi[:, 32:48] + br[:, 32:48] + r * (gh[:, 32:48] + br[:, 48:64]))
        return jnp.maximum((1.0 - z) * n + z * gf, 0.0)

    h1 = graph_branch(t21_ref[...], s21_ref[...], gf1_ref[...])
    h2 = graph_branch(t22_ref[...], s22_ref[...], gf2_ref[...])
    d1 = jnp.sum(h1 * h1, axis=1, keepdims=True)
    d2 = jnp.sum(h2 * h2, axis=1, keepdims=True)
    h = h1 * h2 / (jnp.sqrt(d1) * jnp.sqrt(d2))
    hp = _mm(h, emb_ref[...])
    lane = jax.lax.broadcasted_iota(jnp.int32, hp.shape, 1)
    hp = jnp.where((lane >= G) & (lane < 2 * G), 1.0, hp)
    x = x_ref[...]
    out = _mm(x, W_ref[0]) + b_ref[0]
    out2 = _mm(out, W_ref[1]) + b_ref[1]
    out3 = jnp.tanh(out2)
    out4 = _mm(out3, W_ref[2]) + b_ref[2] + out
    out5 = jnp.tanh(out4)
    out6 = _mm(out5, W_ref[3]) + b_ref[3]
    out7 = jnp.tanh(out6)
    out8 = _mm(out7, W_ref[4]) + b_ref[4] + out5
    o_ref[...] = _mm(hp * out8, W_ref[5]) + b_ref[5]


def _pad128(a, rows=None):
    pads = [(0, 0)] * a.ndim
    pads[-1] = (0, 128 - a.shape[-1])
    if rows is not None:
        pads[0] = (0, rows - a.shape[0])
    return jnp.pad(a, pads)


def _row_spec(blk, width):
    return pl.BlockSpec((blk, width), lambda i: (i, 0))


def _full_spec(shape):
    return pl.BlockSpec(shape, lambda i: (0,) * len(shape))


def kernel(n1, e1, edge_index1, gid1, n2, e2, edge_index2, gid2, x, W_pn, b_pn, W_pe1, b_pe1, W_pe2, b_pe2, W_et, b_et, Wg_ih, Wg_hh, bg_ih, bg_hh, W_cl, b_cl, W_pr, b_pr, Wr_ih, Wr_hh, br_ih, br_hh, W1, b1, W2, b2, W3, b3, W4, b4, W5, b5, Wf, bf):
    f32 = jnp.float32
    Wpnp = jnp.pad(W_pn, ((0, 4), (0, 0)))
    Wpa = jnp.pad(W_pe1[:12], ((0, 4), (0, 0)))
    Wpb = jnp.pad(W_pe1[12:], ((0, 3), (0, 0)))
    u = W_pe2[:16]
    vv = W_pe2[16:]
    c1 = W_cl[:16]
    c2 = W_cl[16:]
    bpn2 = b_pn.reshape(1, G)
    bpe1_2 = b_pe1.reshape(1, G)
    bet2 = b_et.reshape(1, G)
    bpr2 = b_pr.reshape(1, G)
    bcl2 = jnp.reshape(b_cl, (1, 1))

    def gru_pack(W_ih, W_hh, b_ih, b_hh):
        Wg = jnp.concatenate([W_ih.T, W_hh.T], axis=1)
        bg = jnp.concatenate([b_ih[0:16] + b_hh[0:16],
                              b_ih[16:32] + b_hh[16:32],
                              b_ih[32:48], b_hh[32:48]]).reshape(1, 64)
        return Wg, bg

    Wg, bg = gru_pack(Wg_ih, Wg_hh, bg_ih, bg_hh)
    Wr, br = gru_pack(Wr_ih, Wr_hh, br_ih, br_hh)

    def branch(nf, ef, ei, gid):
        src = ei[0]
        dst = ei[1]
        nfp = jnp.pad(nf, ((0, 0), (0, 4)))
        efp = jnp.pad(ef, ((0, 0), (0, 3)))

        hv_new, P, D0 = pl.pallas_call(
            _k1,
            grid=(N // NBLK,),
            in_specs=[_row_spec(NBLK, 16), _full_spec((16, 16)),
                      _full_spec((1, 16)), _full_spec((16, 16)),
                      _full_spec((16, 1))],
            out_specs=[_row_spec(NBLK, 16), _row_spec(NBLK, 16),
                       _row_spec(NBLK, 1)],
            out_shape=[jax.ShapeDtypeStruct((N, G), f32),
                       jax.ShapeDtypeStruct((N, G), f32),
                       jax.ShapeDtypeStruct((N, 1), f32)],
        )(nfp, Wpnp, bpn2, Wpa, u)
        D = D0 + b_pe2[0]

        Psrc = P[src]
        Ddst = D[dst]

        m_e, x_e = pl.pallas_call(
            _k2,
            grid=(E // EBLK,),
            in_specs=[_row_spec(EBLK, 16), _row_spec(EBLK, 1),
                      _row_spec(EBLK, 8), _full_spec((8, 16)),
                      _full_spec((1, 16)), _full_spec((16, 1))],
            out_specs=[_row_spec(EBLK, 16), _row_spec(EBLK, 1)],
            out_shape=[jax.ShapeDtypeStruct((E, G), f32),
                       jax.ShapeDtypeStruct((E, 1), f32)],
        )(Psrc, Ddst, efp, Wpb, bpe1_2, vv)

        t = jax.ops.segment_sum(m_e, dst, num_segments=N)
        s = jax.ops.segment_sum(x_e, dst, num_segments=N)

        hfeat, hv, zn = pl.pallas_call(
            _k4,
            grid=(N // NBLK,),
            in_specs=[_row_spec(NBLK, 16), _row_spec(NBLK, 1),
                      _row_spec(NBLK, 16), _full_spec((16, 16)),
                      _full_spec((1, 16)), _full_spec((16, 96)),
                      _full_spec((1, 64)), _full_spec((16, 16)),
                      _full_spec((1, 16)), _full_spec((16, 1))],
            out_specs=[_row_spec(NBLK, 16), _row_spec(NBLK, 16),
                       _row_spec(NBLK, 1)],
            out_shape=[jax.ShapeDtypeStruct((N, G), f32),
                       jax.ShapeDtypeStruct((N, G), f32),
                       jax.ShapeDtypeStruct((N, 1), f32)],
        )(t, s, hv_new, W_et, bet2, Wg, bg, W_pr, bpr2, c2)

        gid2d = gid.reshape(N, 1)
        gf, dg = pl.pallas_call(
            _k5,
            grid=(N // RBLK,),
            in_specs=[_row_spec(RBLK, 16), _row_spec(RBLK, 1),
                      _full_spec((16, 1)), _full_spec((1, 1))],
            out_specs=[pl.BlockSpec((B, 16), lambda i: (0, 0)),
                       pl.BlockSpec((B, 1), lambda i: (0, 0))],
            out_shape=[jax.ShapeDtypeStruct((B, G), f32),
                       jax.ShapeDtypeStruct((B, 1), f32)],
        )(hfeat, gid2d, c1, bcl2)

        t2, s2 = pl.pallas_call(
            _k6,
            grid=(N // RBLK,),
            in_specs=[_row_spec(RBLK, 16), _row_spec(RBLK, 1),
                      _row_spec(RBLK, 1), _full_spec((B, 1))],
            out_specs=[pl.BlockSpec((B, 16), lambda i: (0, 0)),
                       pl.BlockSpec((B, 1), lambda i: (0, 0))],
            out_shape=[jax.ShapeDtypeStruct((B, G), f32),
                       jax.ShapeDtypeStruct((B, 1), f32)],
        )(hv, zn, gid2d, dg)
        return t2, s2, gf

    t21, s21, gf1 = branch(n1, e1, edge_index1, gid1)
    t22, s22, gf2 = branch(n2, e2, edge_index2, gid2)

    Ws = jnp.stack([_pad128(W1, 128), _pad128(W2, 128), _pad128(W3, 128),
                    _pad128(W4, 128), _pad128(W5, 128), _pad128(Wf, 128)])
    bs = jnp.stack([_pad128(b1), _pad128(b2), _pad128(b3), _pad128(b4),
                    _pad128(b5), _pad128(bf), jnp.zeros((128,), f32),
                    jnp.zeros((128,), f32)])
    emb = jnp.pad(jnp.eye(G, dtype=f32), ((0, 0), (0, 112)))

    out_p = pl.pallas_call(
        _k7,
        out_shape=jax.ShapeDtypeStruct((B, 128), f32),
    )(t21, s21, gf1, t22, s22, gf2, Wr, br, _pad128(x), Ws, bs, emb)
    return out_p[:, :2]


# speedup vs baseline: 4.7922x; 4.7922x over previous
import functools

import jax
import jax.numpy as jnp
from jax import lax
from jax.experimental import pallas as pl
from jax.experimental.pallas import tpu as pltpu
from jax.experimental.pallas import tpu_sc as plsc

N = 50000
E = 800000
B = 2048
G = 16

NBLK = 2000
RBLK = 1000
EPS = 1e-12

NC = 2
NS = 16
NW = NC * NS
EPAD = 819200
EROWS = EPAD // 128
RW = EROWS // NW
CR = 8
NCH = RW // CR
CHS = CR * 128
NPAD = 50048
NROWS_SUB = NPAD // NS
EBLK = 3200


def _leaky(x):
    return jnp.where(x >= 0, x, 0.01 * x)


def _sigmoid(x):
    return 1.0 / (1.0 + jnp.exp(-x))


def _elu(x):
    return jnp.where(x > 0, x, jnp.exp(x) - 1.0)


def _mm(a, b):
    return jax.lax.dot_general(a, b, (((1,), (0,)), ((), ())),
                               preferred_element_type=jnp.float32)


def _k1(nf_ref, Wpn_ref, bpn_ref, Wpa_ref, u_ref, bpe2_ref, o_hv, o_P, o_D):
    nf = nf_ref[...]
    hv = _leaky(_mm(nf, Wpn_ref[...]) + bpn_ref[...])
    o_hv[...] = hv
    o_P[...] = _mm(nf, Wpa_ref[...])
    o_D[...] = _mm(hv, u_ref[...]) + bpe2_ref[0, 0]


def _k2(Ps_ref, Dd_ref, ef_ref, Wpb_ref, bpe1_ref, vv_ref, emb_ref, col_ref,
        o_mx):
    i = pl.program_id(0)
    he1 = _leaky(Ps_ref[...] + _mm(ef_ref[...], Wpb_ref[...]) + bpe1_ref[...])
    lg = Dd_ref[...] + _mm(he1, vv_ref[...])
    ex = jnp.exp(_leaky(lg))
    eid = i * EBLK + jax.lax.broadcasted_iota(jnp.int32, (EBLK, 1), 0)
    ex = jnp.where(eid < E, ex, 0.0)
    o_mx[...] = _mm(ex * he1, emb_ref[...]) + _mm(ex, col_ref[...])


def _sc_gather(P_hbm, D_hbm, src2_hbm, dst1_hbm, o_psrc, o_ddst,
               idx_v, rows_v, dst_v, dd_v, D_v, sem):
    c = lax.axis_index("c")
    s = lax.axis_index("s")
    w = s * NC + c
    rowbase = w * RW
    pltpu.sync_copy(D_hbm, D_v)

    def chunk(g, carry):
        rb = rowbase + g * CR
        eb = rb * 128
        pltpu.sync_copy(src2_hbm.at[pl.ds(rb, CR)], idx_v)
        descs = [
            pltpu.async_copy(P_hbm.at[idx_v.at[j]],
                             rows_v.at[pl.ds(j * 128, 128)], sem)
            for j in range(CR)
        ]
        pltpu.sync_copy(dst1_hbm.at[pl.ds(eb, CHS)], dst_v)
        for d in descs:
            d.wait()
        pltpu.sync_copy(rows_v, o_psrc.at[pl.ds(eb, CHS)])

        def dg(j2, carry2):
            ii = dst_v[pl.ds(j2 * 16, 16)]
            dd_v[pl.ds(j2 * 16, 16)] = plsc.load_gather(D_v, [ii])
            return carry2

        lax.fori_loop(0, CR * 8, dg, 0)
        pltpu.sync_copy(dd_v, o_ddst.at[pl.ds(eb, CHS)])
        return carry

    lax.fori_loop(0, NCH, chunk, 0)


def _sc_scatter(mx_hbm, dst2_hbm, z_hbm, o_acc, mx_v, idx_v, table, sem):
    c = lax.axis_index("c")
    s = lax.axis_index("s")
    w = s * NC + c
    nbase = s * NROWS_SUB
    pltpu.sync_copy(z_hbm, table.at[pl.ds(nbase, NROWS_SUB)])
    plsc.subcore_barrier()

    def chunk(g, carry):
        rb = w * RW + g * CR
        eb = rb * 128
        pltpu.sync_copy(dst2_hbm.at[pl.ds(rb, CR)], idx_v)
        for h in range(2):
            pltpu.sync_copy(mx_hbm.at[pl.ds(eb + h * (CHS // 2), CHS // 2)],
                            mx_v)
            descs = [
                pltpu.async_copy(mx_v.at[pl.ds(j * 128, 128)],
                                 table.at[idx_v.at[h * (CR // 2) + j]],
                                 sem, add=True)
                for j in range(CR // 2)
            ]
            for d in descs:
                d.wait()
        return carry

    lax.fori_loop(0, NCH, chunk, 0)
    plsc.subcore_barrier()
    pltpu.sync_copy(table.at[pl.ds(nbase, NROWS_SUB)],
                    o_acc.at[c].at[pl.ds(nbase, NROWS_SUB)])


def _k4(ts0_ref, ts1_ref, hv_ref, Wet_ref, bet_ref, Wg_ref, bg_ref, Wpr_ref,
        bpr_ref, c2_ref, o_hf, o_hv2, o_zn):
    ts = ts0_ref[...] + ts1_ref[...]
    t = ts[:, 0:16]
    s = ts[:, 16:17]
    occ = s / (s + EPS)
    c = _mm(t / (s + EPS), Wet_ref[...]) + bet_ref[...] * occ
    xg = _elu(c)
    h = hv_ref[...]
    gi = _mm(xg, Wg_ref[:, 0:48])
    gh = _mm(h, Wg_ref[:, 48:96])
    bg = bg_ref[...]
    r = _sigmoid(gi[:, 0:16] + gh[:, 0:16] + bg[:, 0:16])
    z = _sigmoid(gi[:, 16:32] + gh[:, 16:32] + bg[:, 16:32])
    n = jnp.tanh(gi[:, 32:48] + bg[:, 32:48] + r * (gh[:, 32:48] + bg[:, 48:64]))
    hf = jnp.maximum((1.0 - z) * n + z * h, 0.0)
    o_hf[...] = hf
    o_hv2[...] = _mm(hf, Wpr_ref[...]) + bpr_ref[...]
    o_zn[...] = _mm(hf, c2_ref[...])


def _k5(hf_ref, gid_ref, c1_ref, bcl_ref, o_gf, o_dg):
    i = pl.program_id(0)
    gid = gid_ref[...]
    iota = jax.lax.broadcasted_iota(jnp.int32, (RBLK, B), 1)
    oh = jnp.where(gid == iota, 1.0, 0.0)
    contrib = jax.lax.dot_general(oh, hf_ref[...], (((0,), (0,)), ((), ())),
                                  preferred_element_type=jnp.float32)

    @pl.when(i == 0)
    def _():
        o_gf[...] = jnp.zeros_like(o_gf)
        o_dg[...] = jnp.zeros_like(o_dg)

    o_gf[...] += contrib

    @pl.when(i == pl.num_programs(0) - 1)
    def _():
        gf = o_gf[...]
        o_dg[...] = _mm(jnp.maximum(gf, 0.0), c1_ref[...]) + bcl_ref[0, 0]


def _k6(hv_ref, zn_ref, gid_ref, dg_ref, o_t2, o_s2):
    i = pl.program_id(0)
    gid = gid_ref[...]
    iota = jax.lax.broadcasted_iota(jnp.int32, (RBLK, B), 1)
    oh = jnp.where(gid == iota, 1.0, 0.0)
    dgn = _mm(oh, dg_ref[...])
    zl = _leaky(dgn + zn_ref[...])
    ex2 = jnp.exp(zl)
    ct = jax.lax.dot_general(oh, ex2 * hv_ref[...], (((0,), (0,)), ((), ())),
                             preferred_element_type=jnp.float32)
    cs = jax.lax.dot_general(oh, ex2, (((0,), (0,)), ((), ())),
                             preferred_element_type=jnp.float32)

    @pl.when(i == 0)
    def _():
        o_t2[...] = jnp.zeros_like(o_t2)
        o_s2[...] = jnp.zeros_like(o_s2)

    o_t2[...] += ct
    o_s2[...] += cs


def _k7(t21_ref, s21_ref, gf1_ref, t22_ref, s22_ref, gf2_ref, Wr_ref, br_ref,
        x_ref, W_ref, b_ref, emb_ref, o_ref):
    def graph_branch(t2, s2, gf):
        grepr = _elu(t2 / (s2 + EPS))
        gi = _mm(grepr, Wr_ref[:, 0:48])
        gh = _mm(gf, Wr_ref[:, 48:96])
        br = br_ref[...]
        r = _sigmoid(gi[:, 0:16] + gh[:, 0:16] + br[:, 0:16])
        z = _sigmoid(gi[:, 16:32] + gh[:, 16:32] + br[:, 16:32])
        n = jnp.tanh(gi[:, 32:48] + br[:, 32:48] + r * (gh[:, 32:48] + br[:, 48:64]))
        return jnp.maximum((1.0 - z) * n + z * gf, 0.0)

    h1 = graph_branch(t21_ref[...], s21_ref[...], gf1_ref[...])
    h2 = graph_branch(t22_ref[...], s22_ref[...], gf2_ref[...])
    d1 = jnp.sum(h1 * h1, axis=1, keepdims=True)
    d2 = jnp.sum(h2 * h2, axis=1, keepdims=True)
    h = h1 * h2 / (jnp.sqrt(d1) * jnp.sqrt(d2))
    hp = _mm(h, emb_ref[...])
    lane = jax.lax.broadcasted_iota(jnp.int32, hp.shape, 1)
    hp = jnp.where((lane >= G) & (lane < 2 * G), 1.0, hp)
    x = x_ref[...]
    out = _mm(x, W_ref[0]) + b_ref[0]
    out2 = _mm(out, W_ref[1]) + b_ref[1]
    out3 = jnp.tanh(out2)
    out4 = _mm(out3, W_ref[2]) + b_ref[2] + out
    out5 = jnp.tanh(out4)
    out6 = _mm(out5, W_ref[3]) + b_ref[3]
    out7 = jnp.tanh(out6)
    out8 = _mm(out7, W_ref[4]) + b_ref[4] + out5
    o_ref[...] = _mm(hp * out8, W_ref[5]) + b_ref[5]


def _pad128(a, rows=None):
    pads = [(0, 0)] * a.ndim
    pads[-1] = (0, 128 - a.shape[-1])
    if rows is not None:
        pads[0] = (0, rows - a.shape[0])
    return jnp.pad(a, pads)


def _row_spec(blk, width):
    return pl.BlockSpec((blk, width), lambda i: (i, 0))


def _full_spec(shape):
    return pl.BlockSpec(shape, lambda i: (0,) * len(shape))


def kernel(n1, e1, edge_index1, gid1, n2, e2, edge_index2, gid2, x, W_pn, b_pn, W_pe1, b_pe1, W_pe2, b_pe2, W_et, b_et, Wg_ih, Wg_hh, bg_ih, bg_hh, W_cl, b_cl, W_pr, b_pr, Wr_ih, Wr_hh, br_ih, br_hh, W1, b1, W2, b2, W3, b3, W4, b4, W5, b5, Wf, bf):
    f32 = jnp.float32
    Wpnp = jnp.pad(W_pn, ((0, 4), (0, 0)))
    Wpa = jnp.pad(W_pe1[:12], ((0, 4), (0, 0)))
    Wpb = jnp.pad(W_pe1[12:], ((0, 3), (0, 0)))
    u = W_pe2[:16]
    vv = W_pe2[16:]
    c1 = W_cl[:16]
    c2 = W_cl[16:]
    bpn2 = b_pn.reshape(1, G)
    bpe1_2 = b_pe1.reshape(1, G)
    bet2 = b_et.reshape(1, G)
    bpr2 = b_pr.reshape(1, G)
    bcl2 = jnp.reshape(b_cl, (1, 1))
    bpe2_2 = jnp.reshape(b_pe2, (1, 1))

    def gru_pack(W_ih, W_hh, b_ih, b_hh):
        Wg = jnp.concatenate([W_ih.T, W_hh.T], axis=1)
        bg = jnp.concatenate([b_ih[0:16] + b_hh[0:16],
                              b_ih[16:32] + b_hh[16:32],
                              b_ih[32:48], b_hh[32:48]]).reshape(1, 64)
        return Wg, bg

    Wg, bg = gru_pack(Wg_ih, Wg_hh, bg_ih, bg_hh)
    Wr, br = gru_pack(Wr_ih, Wr_hh, br_ih, br_hh)

    mesh = plsc.VectorSubcoreMesh(core_axis_name="c", subcore_axis_name="s")
    zeros_blk = jnp.zeros((NROWS_SUB, 32), f32)
    emb1632 = jnp.pad(jnp.eye(G, dtype=f32), ((0, 0), (0, 16)))
    col16 = jnp.zeros((1, 32), f32).at[0, 16].set(1.0)

    sc_gather = pl.kernel(
        _sc_gather,
        out_type=[jax.ShapeDtypeStruct((EPAD, G), f32),
                  jax.ShapeDtypeStruct((EPAD,), f32)],
        mesh=mesh,
        scratch_types=[pltpu.VMEM((CR, 128), jnp.int32),
                       pltpu.VMEM((CHS, G), f32),
                       pltpu.VMEM((CHS,), jnp.int32),
                       pltpu.VMEM((CHS,), f32),
                       pltpu.VMEM((N,), f32),
                       pltpu.SemaphoreType.DMA],
        compiler_params=pltpu.CompilerParams(needs_layout_passes=False,
                                             use_tc_tiling_on_sc=False),
    )

    sc_scatter = pl.kernel(
        _sc_scatter,
        out_type=jax.ShapeDtypeStruct((NC, NPAD, 32), f32),
        mesh=mesh,
        scratch_types=[pltpu.VMEM((CHS // 2, 32), f32),
                       pltpu.VMEM((CR, 128), jnp.int32),
                       pltpu.VMEM_SHARED((NPAD, 32), f32),
                       pltpu.SemaphoreType.DMA],
        compiler_params=pltpu.CompilerParams(use_tc_tiling_on_sc=False),
    )

    def branch(nf, ef, ei, gid):
        src = ei[0]
        dst = ei[1]
        nfp = jnp.pad(nf, ((0, 0), (0, 4)))
        efp = jnp.pad(ef, ((0, EPAD - E), (0, 3)))
        src2 = jnp.pad(src, (0, EPAD - E)).reshape(EROWS, 128)
        dstp = jnp.pad(dst, (0, EPAD - E))
        dst2 = dstp.reshape(EROWS, 128)

        hv_new, P, D0 = pl.pallas_call(
            _k1,
            grid=(N // NBLK,),
            in_specs=[_row_spec(NBLK, 16), _full_spec((16, 16)),
                      _full_spec((1, 16)), _full_spec((16, 16)),
                      _full_spec((16, 1)), _full_spec((1, 1))],
            out_specs=[_row_spec(NBLK, 16), _row_spec(NBLK, 16),
                       _row_spec(NBLK, 1)],
            out_shape=[jax.ShapeDtypeStruct((N, G), f32),
                       jax.ShapeDtypeStruct((N, G), f32),
                       jax.ShapeDtypeStruct((N, 1), f32)],
        )(nfp, Wpnp, bpn2, Wpa, u, bpe2_2)

        Psrc, Ddst = sc_gather(P, D0.reshape(N), src2, dstp)

        mx = pl.pallas_call(
            _k2,
            grid=(EPAD // EBLK,),
            in_specs=[_row_spec(EBLK, 16), _row_spec(EBLK, 1),
                      _row_spec(EBLK, 8), _full_spec((8, 16)),
                      _full_spec((1, 16)), _full_spec((16, 1)),
                      _full_spec((16, 32)), _full_spec((1, 32))],
            out_specs=_row_spec(EBLK, 32),
            out_shape=jax.ShapeDtypeStruct((EPAD, 32), f32),
        )(Psrc, Ddst.reshape(EPAD, 1), efp, Wpb, bpe1_2, vv, emb1632, col16)

        acc = sc_scatter(mx, dst2, zeros_blk)
        acc = acc[:, :N]

        hfeat, hv, zn = pl.pallas_call(
            _k4,
            grid=(N // NBLK,),
            in_specs=[_row_spec(NBLK, 32), _row_spec(NBLK, 32),
                      _row_spec(NBLK, 16), _full_spec((16, 16)),
                      _full_spec((1, 16)), _full_spec((16, 96)),
                      _full_spec((1, 64)), _full_spec((16, 16)),
                      _full_spec((1, 16)), _full_spec((16, 1))],
            out_specs=[_row_spec(NBLK, 16), _row_spec(NBLK, 16),
                       _row_spec(NBLK, 1)],
            out_shape=[jax.ShapeDtypeStruct((N, G), f32),
                       jax.ShapeDtypeStruct((N, G), f32),
                       jax.ShapeDtypeStruct((N, 1), f32)],
        )(acc[0], acc[1], hv_new, W_et, bet2, Wg, bg, W_pr, bpr2, c2)

        gid2d = gid.reshape(N, 1)
        gf, dg = pl.pallas_call(
            _k5,
            grid=(N // RBLK,),
            in_specs=[_row_spec(RBLK, 16), _row_spec(RBLK, 1),
                      _full_spec((16, 1)), _full_spec((1, 1))],
            out_specs=[pl.BlockSpec((B, 16), lambda i: (0, 0)),
                       pl.BlockSpec((B, 1), lambda i: (0, 0))],
            out_shape=[jax.ShapeDtypeStruct((B, G), f32),
                       jax.ShapeDtypeStruct((B, 1), f32)],
        )(hfeat, gid2d, c1, bcl2)

        t2, s2 = pl.pallas_call(
            _k6,
            grid=(N // RBLK,),
            in_specs=[_row_spec(RBLK, 16), _row_spec(RBLK, 1),
                      _row_spec(RBLK, 1), _full_spec((B, 1))],
            out_specs=[pl.BlockSpec((B, 16), lambda i: (0, 0)),
                       pl.BlockSpec((B, 1), lambda i: (0, 0))],
            out_shape=[jax.ShapeDtypeStruct((B, G), f32),
                       jax.ShapeDtypeStruct((B, 1), f32)],
        )(hv, zn, gid2d, dg)
        return t2, s2, gf

    t21, s21, gf1 = branch(n1, e1, edge_index1, gid1)
    t22, s22, gf2 = branch(n2, e2, edge_index2, gid2)

    Ws = jnp.stack([_pad128(W1, 128), _pad128(W2, 128), _pad128(W3, 128),
                    _pad128(W4, 128), _pad128(W5, 128), _pad128(Wf, 128)])
    bs = jnp.stack([_pad128(b1), _pad128(b2), _pad128(b3), _pad128(b4),
                    _pad128(b5), _pad128(bf), jnp.zeros((128,), f32),
                    jnp.zeros((128,), f32)])
    emb = jnp.pad(jnp.eye(G, dtype=f32), ((0, 0), (0, 112)))

    out_p = pl.pallas_call(
        _k7,
        out_shape=jax.ShapeDtypeStruct((B, 128), f32),
    )(t21, s21, gf1, t22, s22, gf2, Wr, br, _pad128(x), Ws, bs, emb)
    return out_p[:, :2]


# speedup vs baseline: 4.9836x; 1.0399x over previous
import functools

import jax
import jax.numpy as jnp
from jax import lax
from jax.experimental import pallas as pl
from jax.experimental.pallas import tpu as pltpu
from jax.experimental.pallas import tpu_sc as plsc

N = 50000
E = 800000
B = 2048
G = 16

NBLK = 2000
RBLK = 1000
EPS = 1e-12

NC = 2
NS = 16
NW = NC * NS
EPAD = 819200
EROWS = EPAD // 128
RW = EROWS // NW
CR = 8
NCH = RW // CR
CHS = CR * 128
NPAD = 50048
NROWS_SUB = NPAD // NS
EBLK = 6400


def _leaky(x):
    return jnp.where(x >= 0, x, 0.01 * x)


def _sigmoid(x):
    return 1.0 / (1.0 + jnp.exp(-x))


def _elu(x):
    return jnp.where(x > 0, x, jnp.exp(x) - 1.0)


def _mm(a, b):
    return jax.lax.dot_general(a, b, (((1,), (0,)), ((), ())),
                               preferred_element_type=jnp.float32)


def _k1(nf_ref, Wpn_ref, bpn_ref, Wpa_ref, u_ref, bpe2_ref, o_hv, o_P, o_D):
    nf = nf_ref[...]
    hv = _leaky(_mm(nf, Wpn_ref[...]) + bpn_ref[...])
    o_hv[...] = hv
    o_P[...] = _mm(nf, Wpa_ref[...])
    o_D[...] = _mm(hv, u_ref[...]) + bpe2_ref[0, 0]


def _k2(Ps_ref, Dd_ref, ef_ref, Wpb_ref, bpe1_ref, vv_ref, emb_ref, col_ref,
        o_mx):
    i = pl.program_id(0)
    he1 = _leaky(Ps_ref[...] + _mm(ef_ref[...], Wpb_ref[...]) + bpe1_ref[...])
    lg = Dd_ref[...] + _mm(he1, vv_ref[...])
    ex = jnp.exp(_leaky(lg))
    eid = i * EBLK + jax.lax.broadcasted_iota(jnp.int32, (EBLK, 1), 0)
    ex = jnp.where(eid < E, ex, 0.0)
    o_mx[...] = _mm(ex * he1, emb_ref[...]) + ex * col_ref[...]


def _sc_gather(P_hbm, D_hbm, src2_hbm, dst1_hbm, o_psrc, o_ddst,
               idx_v, rows_v, dst_v, dd_v, D_v, sem):
    c = lax.axis_index("c")
    s = lax.axis_index("s")
    w = s * NC + c
    rowbase = w * RW
    pltpu.sync_copy(D_hbm, D_v)

    def chunk(g, carry):
        rb = rowbase + g * CR
        eb = rb * 128
        pltpu.sync_copy(src2_hbm.at[pl.ds(rb, CR)], idx_v)
        descs = [
            pltpu.async_copy(P_hbm.at[idx_v.at[j]],
                             rows_v.at[pl.ds(j * 128, 128)], sem)
            for j in range(CR)
        ]
        pltpu.sync_copy(dst1_hbm.at[pl.ds(eb, CHS)], dst_v)
        for d in descs:
            d.wait()
        pltpu.sync_copy(rows_v, o_psrc.at[pl.ds(eb, CHS)])

        def dg(j2, carry2):
            ii = dst_v[pl.ds(j2 * 16, 16)]
            dd_v[pl.ds(j2 * 16, 16)] = plsc.load_gather(D_v, [ii])
            return carry2

        lax.fori_loop(0, CR * 8, dg, 0)
        pltpu.sync_copy(dd_v, o_ddst.at[pl.ds(eb, CHS)])
        return carry

    lax.fori_loop(0, NCH, chunk, 0)


def _sc_scatter(mx_hbm, dst2_hbm, z_hbm, o_acc, mx_v, idx_v, table, sem):
    c = lax.axis_index("c")
    s = lax.axis_index("s")
    w = s * NC + c
    nbase = s * NROWS_SUB
    pltpu.sync_copy(z_hbm, table.at[pl.ds(nbase, NROWS_SUB)])
    plsc.subcore_barrier()

    def chunk(g, carry):
        rb = w * RW + g * CR
        eb = rb * 128
        pltpu.sync_copy(dst2_hbm.at[pl.ds(rb, CR)], idx_v)
        for h in range(2):
            pltpu.sync_copy(mx_hbm.at[pl.ds(eb + h * (CHS // 2), CHS // 2)],
                            mx_v)
            descs = [
                pltpu.async_copy(mx_v.at[pl.ds(j * 128, 128)],
                                 table.at[idx_v.at[h * (CR // 2) + j]],
                                 sem, add=True)
                for j in range(CR // 2)
            ]
            for d in descs:
                d.wait()
        return carry

    lax.fori_loop(0, NCH, chunk, 0)
    plsc.subcore_barrier()
    pltpu.sync_copy(table.at[pl.ds(nbase, NROWS_SUB)],
                    o_acc.at[c].at[pl.ds(nbase, NROWS_SUB)])


def _k4(ts0_ref, ts1_ref, hv_ref, Wet_ref, bet_ref, Wg_ref, bg_ref, Wpr_ref,
        bpr_ref, c2_ref, o_hf, o_hv2, o_zn):
    ts = ts0_ref[...] + ts1_ref[...]
    t = ts[:, 0:16]
    s = ts[:, 16:17]
    occ = s / (s + EPS)
    c = _mm(t / (s + EPS), Wet_ref[...]) + bet_ref[...] * occ
    xg = _elu(c)
    h = hv_ref[...]
    gi = _mm(xg, Wg_ref[:, 0:48])
    gh = _mm(h, Wg_ref[:, 48:96])
    bg = bg_ref[...]
    r = _sigmoid(gi[:, 0:16] + gh[:, 0:16] + bg[:, 0:16])
    z = _sigmoid(gi[:, 16:32] + gh[:, 16:32] + bg[:, 16:32])
    n = jnp.tanh(gi[:, 32:48] + bg[:, 32:48] + r * (gh[:, 32:48] + bg[:, 48:64]))
    hf = jnp.maximum((1.0 - z) * n + z * h, 0.0)
    o_hf[...] = hf
    o_hv2[...] = _mm(hf, Wpr_ref[...]) + bpr_ref[...]
    o_zn[...] = _mm(hf, c2_ref[...])


def _k5(hf_ref, gid_ref, c1_ref, bcl_ref, o_gf, o_dg):
    i = pl.program_id(0)
    gid = gid_ref[...]
    iota = jax.lax.broadcasted_iota(jnp.int32, (RBLK, B), 1)
    oh = jnp.where(gid == iota, 1.0, 0.0)
    contrib = jax.lax.dot_general(oh, hf_ref[...], (((0,), (0,)), ((), ())),
                                  preferred_element_type=jnp.float32)

    @pl.when(i == 0)
    def _():
        o_gf[...] = jnp.zeros_like(o_gf)
        o_dg[...] = jnp.zeros_like(o_dg)

    o_gf[...] += contrib

    @pl.when(i == pl.num_programs(0) - 1)
    def _():
        gf = o_gf[...]
        o_dg[...] = _mm(jnp.maximum(gf, 0.0), c1_ref[...]) + bcl_ref[0, 0]


def _k6(hv_ref, zn_ref, gid_ref, dg_ref, o_t2, o_s2):
    i = pl.program_id(0)
    gid = gid_ref[...]
    iota = jax.lax.broadcasted_iota(jnp.int32, (RBLK, B), 1)
    oh = jnp.where(gid == iota, 1.0, 0.0)
    dgn = _mm(oh, dg_ref[...])
    zl = _leaky(dgn + zn_ref[...])
    ex2 = jnp.exp(zl)
    ct = jax.lax.dot_general(oh, ex2 * hv_ref[...], (((0,), (0,)), ((), ())),
                             preferred_element_type=jnp.float32)
    cs = jax.lax.dot_general(oh, ex2, (((0,), (0,)), ((), ())),
                             preferred_element_type=jnp.float32)

    @pl.when(i == 0)
    def _():
        o_t2[...] = jnp.zeros_like(o_t2)
        o_s2[...] = jnp.zeros_like(o_s2)

    o_t2[...] += ct
    o_s2[...] += cs


def _k7(t21_ref, s21_ref, gf1_ref, t22_ref, s22_ref, gf2_ref, Wr_ref, br_ref,
        x_ref, W_ref, b_ref, emb_ref, o_ref):
    def graph_branch(t2, s2, gf):
        grepr = _elu(t2 / (s2 + EPS))
        gi = _mm(grepr, Wr_ref[:, 0:48])
        gh = _mm(gf, Wr_ref[:, 48:96])
        br = br_ref[...]
        r = _sigmoid(gi[:, 0:16] + gh[:, 0:16] + br[:, 0:16])
        z = _sigmoid(gi[:, 16:32] + gh[:, 16:32] + br[:, 16:32])
        n = jnp.tanh(gi[:, 32:48] + br[:, 32:48] + r * (gh[:, 32:48] + br[:, 48:64]))
        return jnp.maximum((1.0 - z) * n + z * gf, 0.0)

    h1 = graph_branch(t21_ref[...], s21_ref[...], gf1_ref[...])
    h2 = graph_branch(t22_ref[...], s22_ref[...], gf2_ref[...])
    d1 = jnp.sum(h1 * h1, axis=1, keepdims=True)
    d2 = jnp.sum(h2 * h2, axis=1, keepdims=True)
    h = h1 * h2 / (jnp.sqrt(d1) * jnp.sqrt(d2))
    hp = _mm(h, emb_ref[...])
    lane = jax.lax.broadcasted_iota(jnp.int32, hp.shape, 1)
    hp = jnp.where((lane >= G) & (lane < 2 * G), 1.0, hp)
    x = x_ref[...]
    out = _mm(x, W_ref[0]) + b_ref[0]
    out2 = _mm(out, W_ref[1]) + b_ref[1]
    out3 = jnp.tanh(out2)
    out4 = _mm(out3, W_ref[2]) + b_ref[2] + out
    out5 = jnp.tanh(out4)
    out6 = _mm(out5, W_ref[3]) + b_ref[3]
    out7 = jnp.tanh(out6)
    out8 = _mm(out7, W_ref[4]) + b_ref[4] + out5
    o_ref[...] = _mm(hp * out8, W_ref[5]) + b_ref[5]


def _pad128(a, rows=None):
    pads = [(0, 0)] * a.ndim
    pads[-1] = (0, 128 - a.shape[-1])
    if rows is not None:
        pads[0] = (0, rows - a.shape[0])
    return jnp.pad(a, pads)


def _row_spec(blk, width):
    return pl.BlockSpec((blk, width), lambda i: (i, 0))


def _full_spec(shape):
    return pl.BlockSpec(shape, lambda i: (0,) * len(shape))


def kernel(n1, e1, edge_index1, gid1, n2, e2, edge_index2, gid2, x, W_pn, b_pn, W_pe1, b_pe1, W_pe2, b_pe2, W_et, b_et, Wg_ih, Wg_hh, bg_ih, bg_hh, W_cl, b_cl, W_pr, b_pr, Wr_ih, Wr_hh, br_ih, br_hh, W1, b1, W2, b2, W3, b3, W4, b4, W5, b5, Wf, bf):
    f32 = jnp.float32
    Wpnp = jnp.pad(W_pn, ((0, 4), (0, 0)))
    Wpa = jnp.pad(W_pe1[:12], ((0, 4), (0, 0)))
    Wpb = jnp.pad(W_pe1[12:], ((0, 3), (0, 0)))
    u = W_pe2[:16]
    vv = W_pe2[16:]
    c1 = W_cl[:16]
    c2 = W_cl[16:]
    bpn2 = b_pn.reshape(1, G)
    bpe1_2 = b_pe1.reshape(1, G)
    bet2 = b_et.reshape(1, G)
    bpr2 = b_pr.reshape(1, G)
    bcl2 = jnp.reshape(b_cl, (1, 1))
    bpe2_2 = jnp.reshape(b_pe2, (1, 1))

    def gru_pack(W_ih, W_hh, b_ih, b_hh):
        Wg = jnp.concatenate([W_ih.T, W_hh.T], axis=1)
        bg = jnp.concatenate([b_ih[0:16] + b_hh[0:16],
                              b_ih[16:32] + b_hh[16:32],
                              b_ih[32:48], b_hh[32:48]]).reshape(1, 64)
        return Wg, bg

    Wg, bg = gru_pack(Wg_ih, Wg_hh, bg_ih, bg_hh)
    Wr, br = gru_pack(Wr_ih, Wr_hh, br_ih, br_hh)

    mesh = plsc.VectorSubcoreMesh(core_axis_name="c", subcore_axis_name="s")
    zeros_blk = jnp.zeros((NROWS_SUB, 32), f32)
    emb1632 = jnp.pad(jnp.eye(G, dtype=f32), ((0, 0), (0, 16)))
    col16 = jnp.zeros((1, 32), f32).at[0, 16].set(1.0)

    sc_gather = pl.kernel(
        _sc_gather,
        out_type=[jax.ShapeDtypeStruct((EPAD, G), f32),
                  jax.ShapeDtypeStruct((EPAD,), f32)],
        mesh=mesh,
        scratch_types=[pltpu.VMEM((CR, 128), jnp.int32),
                       pltpu.VMEM((CHS, G), f32),
                       pltpu.VMEM((CHS,), jnp.int32),
                       pltpu.VMEM((CHS,), f32),
                       pltpu.VMEM((N,), f32),
                       pltpu.SemaphoreType.DMA],
        compiler_params=pltpu.CompilerParams(needs_layout_passes=False,
                                             use_tc_tiling_on_sc=False),
    )

    sc_scatter = pl.kernel(
        _sc_scatter,
        out_type=jax.ShapeDtypeStruct((NC, NPAD, 32), f32),
        mesh=mesh,
        scratch_types=[pltpu.VMEM((CHS // 2, 32), f32),
                       pltpu.VMEM((CR, 128), jnp.int32),
                       pltpu.VMEM_SHARED((NPAD, 32), f32),
                       pltpu.SemaphoreType.DMA],
        compiler_params=pltpu.CompilerParams(use_tc_tiling_on_sc=False),
    )

    def branch(nf, ef, ei, gid):
        src = ei[0]
        dst = ei[1]
        nfp = jnp.pad(nf, ((0, 0), (0, 4)))
        efp = jnp.pad(ef, ((0, EPAD - E), (0, 3)))
        src2 = jnp.pad(src, (0, EPAD - E)).reshape(EROWS, 128)
        dstp = jnp.pad(dst, (0, EPAD - E))
        dst2 = dstp.reshape(EROWS, 128)

        hv_new, P, D0 = pl.pallas_call(
            _k1,
            grid=(N // NBLK,),
            in_specs=[_row_spec(NBLK, 16), _full_spec((16, 16)),
                      _full_spec((1, 16)), _full_spec((16, 16)),
                      _full_spec((16, 1)), _full_spec((1, 1))],
            out_specs=[_row_spec(NBLK, 16), _row_spec(NBLK, 16),
                       _row_spec(NBLK, 1)],
            out_shape=[jax.ShapeDtypeStruct((N, G), f32),
                       jax.ShapeDtypeStruct((N, G), f32),
                       jax.ShapeDtypeStruct((N, 1), f32)],
        )(nfp, Wpnp, bpn2, Wpa, u, bpe2_2)

        Psrc, Ddst = sc_gather(P, D0.reshape(N), src2, dstp)

        mx = pl.pallas_call(
            _k2,
            grid=(EPAD // EBLK,),
            in_specs=[_row_spec(EBLK, 16), _row_spec(EBLK, 1),
                      _row_spec(EBLK, 8), _full_spec((8, 16)),
                      _full_spec((1, 16)), _full_spec((16, 1)),
                      _full_spec((16, 32)), _full_spec((1, 32))],
            out_specs=_row_spec(EBLK, 32),
            out_shape=jax.ShapeDtypeStruct((EPAD, 32), f32),
        )(Psrc, Ddst.reshape(EPAD, 1), efp, Wpb, bpe1_2, vv, emb1632, col16)

        acc = sc_scatter(mx, dst2, zeros_blk)
        acc = acc[:, :N]

        hfeat, hv, zn = pl.pallas_call(
            _k4,
            grid=(N // NBLK,),
            in_specs=[_row_spec(NBLK, 32), _row_spec(NBLK, 32),
                      _row_spec(NBLK, 16), _full_spec((16, 16)),
                      _full_spec((1, 16)), _full_spec((16, 96)),
                      _full_spec((1, 64)), _full_spec((16, 16)),
                      _full_spec((1, 16)), _full_spec((16, 1))],
            out_specs=[_row_spec(NBLK, 16), _row_spec(NBLK, 16),
                       _row_spec(NBLK, 1)],
            out_shape=[jax.ShapeDtypeStruct((N, G), f32),
                       jax.ShapeDtypeStruct((N, G), f32),
                       jax.ShapeDtypeStruct((N, 1), f32)],
        )(acc[0], acc[1], hv_new, W_et, bet2, Wg, bg, W_pr, bpr2, c2)

        gid2d = gid.reshape(N, 1)
        gf, dg = pl.pallas_call(
            _k5,
            grid=(N // RBLK,),
            in_specs=[_row_spec(RBLK, 16), _row_spec(RBLK, 1),
                      _full_spec((16, 1)), _full_spec((1, 1))],
            out_specs=[pl.BlockSpec((B, 16), lambda i: (0, 0)),
                       pl.BlockSpec((B, 1), lambda i: (0, 0))],
            out_shape=[jax.ShapeDtypeStruct((B, G), f32),
                       jax.ShapeDtypeStruct((B, 1), f32)],
        )(hfeat, gid2d, c1, bcl2)

        t2, s2 = pl.pallas_call(
            _k6,
            grid=(N // RBLK,),
            in_specs=[_row_spec(RBLK, 16), _row_spec(RBLK, 1),
                      _row_spec(RBLK, 1), _full_spec((B, 1))],
            out_specs=[pl.BlockSpec((B, 16), lambda i: (0, 0)),
                       pl.BlockSpec((B, 1), lambda i: (0, 0))],
            out_shape=[jax.ShapeDtypeStruct((B, G), f32),
                       jax.ShapeDtypeStruct((B, 1), f32)],
        )(hv, zn, gid2d, dg)
        return t2, s2, gf

    t21, s21, gf1 = branch(n1, e1, edge_index1, gid1)
    t22, s22, gf2 = branch(n2, e2, edge_index2, gid2)

    Ws = jnp.stack([_pad128(W1, 128), _pad128(W2, 128), _pad128(W3, 128),
                    _pad128(W4, 128), _pad128(W5, 128), _pad128(Wf, 128)])
    bs = jnp.stack([_pad128(b1), _pad128(b2), _pad128(b3), _pad128(b4),
                    _pad128(b5), _pad128(bf), jnp.zeros((128,), f32),
                    jnp.zeros((128,), f32)])
    emb = jnp.pad(jnp.eye(G, dtype=f32), ((0, 0), (0, 112)))

    out_p = pl.pallas_call(
        _k7,
        out_shape=jax.ShapeDtypeStruct((B, 128), f32),
    )(t21, s21, gf1, t22, s22, gf2, Wr, br, _pad128(x), Ws, bs, emb)
    return out_p[:, :2]
